# baseline (speedup 1.0000x reference)
import functools

import numpy as np
import jax
import jax.numpy as jnp
from jax import lax
from jax.experimental import pallas as pl
from jax.experimental.pallas import tpu as pltpu
from jax.experimental.pallas import tpu_sc as plsc

B = 4
N = 2048
NL = 4
R2 = 2 * N
W = 512
TINY = np.float32(np.finfo(np.float32).tiny)

ROT = ((13, 15, 26, 6), (17, 29, 16, 24))


def _np_threefry_pair(k0, k1, x0, x1):
    with np.errstate(over="ignore"):
        ks = (np.uint32(k0), np.uint32(k1),
              np.uint32(k0) ^ np.uint32(k1) ^ np.uint32(0x1BD11BDA))
        x0 = np.uint32(x0) + ks[0]
        x1 = np.uint32(x1) + ks[1]
        for i in range(5):
            for r in ROT[i % 2]:
                x0 = np.uint32(x0 + x1)
                x1 = np.uint32((np.uint32(x1 << np.uint32(r))
                                | np.uint32(x1 >> np.uint32(32 - r))) ^ x0)
            x0 = np.uint32(x0 + ks[(i + 1) % 3])
            x1 = np.uint32(x1 + ks[(i + 2) % 3] + np.uint32(i + 1))
    return x0, x1


def _np_fold_in(key, data):
    return _np_threefry_pair(key[0], key[1], 0, data)


_KEY42 = (np.uint32(0), np.uint32(42))
KEYS_C = np.stack([_np_fold_in(_KEY42, 2 * i) for i in range(4)]
                  ).astype(np.uint32)
KEYS_B = np.stack([_np_fold_in(_KEY42, 2 * i + 1) for i in range(4)]
                  ).astype(np.uint32)


def _tf_bits(k0, k1, x1):
    ks2 = k0 ^ k1 ^ jnp.uint32(0x1BD11BDA)
    ks = (k0, k1, ks2)
    x1 = x1 + k1
    x0 = None
    for i in range(5):
        for r in ROT[i % 2]:
            x0 = (x1 + k0) if x0 is None else (x0 + x1)
            x1 = ((x1 << jnp.uint32(r)) | (x1 >> jnp.uint32(32 - r))) ^ x0
        x0 = x0 + ks[(i + 1) % 3]
        x1 = x1 + (ks[(i + 2) % 3] + jnp.uint32(i + 1))
    return x0 ^ x1


def _bits_to_unit_float(bits):
    f = lax.bitcast_convert_type(
        (bits >> jnp.uint32(9)) | jnp.uint32(0x3F800000), jnp.float32)
    return f - jnp.float32(1.0)


RT = 128
NT = R2 // RT


def _sample_body(keys_ref, adj_ref, out_ref):
    t = pl.program_id(0)
    k0 = keys_ref[0]
    k1 = keys_ref[1]
    logits = adj_ref[...]
    iota_r = lax.broadcasted_iota(jnp.uint32, (RT, N), 0)
    iota_c = lax.broadcasted_iota(jnp.uint32, (RT, N), 1)
    row0 = (t * RT).astype(jnp.uint32)
    base = (iota_r + row0) * jnp.uint32(N) + iota_c
    iota_ci = lax.broadcasted_iota(jnp.int32, (RT, N), 1)
    cols = []
    for b in range(B):
        bits = _tf_bits(k0, k1, base + jnp.uint32(b * R2 * N))
        u = _bits_to_unit_float(bits)
        uu = jnp.maximum(TINY, u + TINY)
        g = -jnp.log(-jnp.log(uu))
        vals = g + logits
        m = jnp.max(vals, axis=1, keepdims=True)
        idx = jnp.min(jnp.where(vals == m, iota_ci, jnp.int32(N)), axis=1)
        cols.append(idx.reshape(RT, 1))
    out_ref[...] = jnp.concatenate(cols, axis=1)


def _sample_layer(keys_row, adj2):
    return pl.pallas_call(
        _sample_body,
        grid=(NT,),
        in_specs=[
            pl.BlockSpec(memory_space=pltpu.SMEM),
            pl.BlockSpec((RT, N), lambda t: (t, 0)),
        ],
        out_specs=pl.BlockSpec((RT, B), lambda t: (t, 0)),
        out_shape=jax.ShapeDtypeStruct((R2, B), jnp.int32),
    )(keys_row, adj2)


def _bern_body(keys_ref, p_ref, minv_ref):
    l = pl.program_id(0)
    k0 = keys_ref[l, 0]
    k1 = keys_ref[l, 1]
    p = p_ref[0]
    iota_b = lax.broadcasted_iota(jnp.uint32, (B, N), 0)
    iota_c = lax.broadcasted_iota(jnp.uint32, (B, N), 1)
    f = iota_b * jnp.uint32(N) + iota_c
    u = jnp.maximum(jnp.float32(0.0), _bits_to_unit_float(_tf_bits(k0, k1, f)))
    minv_ref[0] = jnp.where(u < p, jnp.int32(-1), jnp.int32(0))


def _bern_all(keys, p_stack):
    return pl.pallas_call(
        _bern_body,
        grid=(NL,),
        in_specs=[
            pl.BlockSpec(memory_space=pltpu.SMEM),
            pl.BlockSpec((1, 1, N), lambda l: (l, 0, 0)),
        ],
        out_specs=pl.BlockSpec((1, B, N), lambda l: (l, 0, 0)),
        out_shape=jax.ShapeDtypeStruct((NL, B, N), jnp.int32),
    )(keys, p_stack)


NSUB = 16
GATES = B * N
GPS = GATES // 2
GPW = GPS // NSUB
G = 32
NCH = GPW // G


def _gather_layer_body(tab, idx_hbm, minv_hbm, out,
                       idx_v0, idx_v1, rows_v0, rows_v1, minv_v, out_v,
                       sem0, sem1):
    sc = lax.axis_index("c")
    sub = lax.axis_index("s")
    gbase = sc * GPS + sub * GPW
    idxv = (idx_v0, idx_v1)
    rowsv = (rows_v0, rows_v1)
    sems = (sem0, sem1)

    def start(c, par):
        pltpu.sync_copy(idx_hbm.at[pl.ds((gbase + c * G) * 2, 2 * G)],
                        idxv[par])
        return pltpu.async_copy(tab.at[idxv[par]], rowsv[par], sems[par])

    def do_chunk(base, par):
        pltpu.sync_copy(minv_hbm.at[pl.ds(base, G)], minv_v)
        pltpu.make_async_copy(tab.at[idxv[par]], rowsv[par], sems[par]).wait()
        rows = rowsv[par]

        def gate(g, carry2):
            m = minv_v[g]
            for cc in range(W // 16):
                a = rows[2 * g, cc * 16:(cc + 1) * 16]
                b = rows[2 * g + 1, cc * 16:(cc + 1) * 16]
                out_v[g, cc * 16:(cc + 1) * 16] = ~((a & b) ^ (m & (a ^ b)))
            return carry2

        lax.fori_loop(0, G, gate, 0)
        pltpu.sync_copy(out_v, out.at[pl.ds(base, G)])

    start(0, 0)

    def pair(cp, carry):
        c0 = 2 * cp
        start(c0 + 1, 1)
        do_chunk(gbase + c0 * G, 0)

        @pl.when(cp + 1 < NCH // 2)
        def _():
            start(c0 + 2, 0)

        do_chunk(gbase + (c0 + 1) * G, 1)
        return carry

    lax.fori_loop(0, NCH // 2, pair, 0)


def _sc_gather_layer(table, idx, minv_sp):
    mesh = plsc.VectorSubcoreMesh(core_axis_name="c", subcore_axis_name="s",
                                  num_cores=2, num_subcores=16)
    return pl.kernel(
        _gather_layer_body,
        out_type=jax.ShapeDtypeStruct((GATES, W), jnp.int32),
        mesh=mesh,
        scratch_types=[
            pltpu.VMEM((2 * G,), jnp.int32),
            pltpu.VMEM((2 * G,), jnp.int32),
            pltpu.VMEM((2 * G, W), jnp.int32),
            pltpu.VMEM((2 * G, W), jnp.int32),
            pltpu.VMEM((G, 16), jnp.int32),
            pltpu.VMEM((G, W), jnp.int32),
            pltpu.SemaphoreType.DMA,
            pltpu.SemaphoreType.DMA,
        ],
    )(table, idx, minv_sp)


def kernel(input_bitarrays, batch_size,
           adj_logits_0, invert_logits_0, adj_logits_1, invert_logits_1,
           adj_logits_2, invert_logits_2, adj_logits_3, invert_logits_3):
    adjs = (adj_logits_0, adj_logits_1, adj_logits_2, adj_logits_3)
    vs = (invert_logits_0, invert_logits_1, invert_logits_2, invert_logits_3)

    keys_c = jnp.asarray(KEYS_C)
    keys_b = jnp.asarray(KEYS_B)

    p_stack = jnp.stack([jax.nn.sigmoid(v) for v in vs]).reshape(NL, 1, N)

    samples = [_sample_layer(keys_c[l], adjs[l].reshape(R2, N))
               for l in range(NL)]
    minv = _bern_all(keys_b, p_stack)

    conn_all = jnp.transpose(jnp.stack(samples).reshape(NL, 2, N, B),
                             (0, 3, 2, 1))
    conns = [conn_all[l] for l in range(NL)]
    invs = [minv[l] != 0 for l in range(NL)]

    boff = (jnp.arange(B, dtype=jnp.int32) * N)[None, :, None, None]
    lsel = jnp.array([0, 1, 1, 1], jnp.int32)[:, None, None, None]
    idx_all = (conn_all + lsel * boff).reshape(NL, 2 * GATES)
    minv_all = jnp.broadcast_to(minv.reshape(NL, GATES, 1), (NL, GATES, 16))

    table = input_bitarrays + (batch_size - B)
    for l in range(NL):
        table = _sc_gather_layer(table, idx_all[l], minv_all[l])

    x = table.reshape(B, N, W)
    return (x,) + tuple(conns) + tuple(invs)


# speedup vs baseline: 1.0383x; 1.0383x over previous
import functools

import numpy as np
import jax
import jax.numpy as jnp
from jax import lax
from jax.experimental import pallas as pl
from jax.experimental.pallas import tpu as pltpu
from jax.experimental.pallas import tpu_sc as plsc

B = 4
N = 2048
NL = 4
R2 = 2 * N
W = 512
TINY = np.float32(np.finfo(np.float32).tiny)

ROT = ((13, 15, 26, 6), (17, 29, 16, 24))


def _tf_bits(k0, k1, x1):
    ks2 = k0 ^ k1 ^ jnp.uint32(0x1BD11BDA)
    ks = (k0, k1, ks2)
    x1 = x1 + k1
    x0 = None
    for i in range(5):
        for r in ROT[i % 2]:
            x0 = (x1 + k0) if x0 is None else (x0 + x1)
            x1 = ((x1 << jnp.uint32(r)) | (x1 >> jnp.uint32(32 - r))) ^ x0
        x0 = x0 + ks[(i + 1) % 3]
        x1 = x1 + (ks[(i + 2) % 3] + jnp.uint32(i + 1))
    return x0 ^ x1


def _bits_to_unit_float(bits):
    f = lax.bitcast_convert_type(
        (bits >> jnp.uint32(9)) | jnp.uint32(0x3F800000), jnp.float32)
    return f - jnp.float32(1.0)


RT = 128
NT = R2 // RT


def _sample_body(keys_ref, adj_ref, out_ref):
    t = pl.program_id(0)
    k0 = keys_ref[0]
    k1 = keys_ref[1]
    logits = adj_ref[...]
    iota_r = lax.broadcasted_iota(jnp.uint32, (RT, N), 0)
    iota_c = lax.broadcasted_iota(jnp.uint32, (RT, N), 1)
    row0 = (t * RT).astype(jnp.uint32)
    base = (iota_r + row0) * jnp.uint32(N) + iota_c
    iota_ci = lax.broadcasted_iota(jnp.int32, (RT, N), 1)
    cols = []
    for b in range(B):
        bits = _tf_bits(k0, k1, base + jnp.uint32(b * R2 * N))
        u = _bits_to_unit_float(bits)
        uu = jnp.maximum(TINY, u + TINY)
        g = -jnp.log(-jnp.log(uu))
        vals = g + logits
        m = jnp.max(vals, axis=1, keepdims=True)
        idx = jnp.min(jnp.where(vals == m, iota_ci, jnp.int32(N)), axis=1)
        cols.append(idx.reshape(RT, 1))
    out_ref[...] = jnp.concatenate(cols, axis=1)


def _sample_layer(keys_row, adj2):
    return pl.pallas_call(
        _sample_body,
        grid=(NT,),
        in_specs=[
            pl.BlockSpec(memory_space=pltpu.SMEM),
            pl.BlockSpec((RT, N), lambda t: (t, 0)),
        ],
        out_specs=pl.BlockSpec((RT, B), lambda t: (t, 0)),
        out_shape=jax.ShapeDtypeStruct((R2, B), jnp.int32),
    )(keys_row, adj2)


def _bern_body(keys_ref, p_ref, minv_ref):
    l = pl.program_id(0)
    k0 = keys_ref[l, 0]
    k1 = keys_ref[l, 1]
    p = p_ref[0]
    iota_b = lax.broadcasted_iota(jnp.uint32, (B, N), 0)
    iota_c = lax.broadcasted_iota(jnp.uint32, (B, N), 1)
    f = iota_b * jnp.uint32(N) + iota_c
    u = jnp.maximum(jnp.float32(0.0), _bits_to_unit_float(_tf_bits(k0, k1, f)))
    minv_ref[0] = jnp.where(u < p, jnp.int32(-1), jnp.int32(0))


def _bern_all(keys, p_stack):
    return pl.pallas_call(
        _bern_body,
        grid=(NL,),
        in_specs=[
            pl.BlockSpec(memory_space=pltpu.SMEM),
            pl.BlockSpec((1, 1, N), lambda l: (l, 0, 0)),
        ],
        out_specs=pl.BlockSpec((1, B, N), lambda l: (l, 0, 0)),
        out_shape=jax.ShapeDtypeStruct((NL, B, N), jnp.int32),
    )(keys, p_stack)


NSUB = 16
GATES = B * N
GPS = GATES // 2
GPW = GPS // NSUB
G = 32
NCH = GPW // G


def _gather_layer_body(tab, idx_hbm, minv_hbm, out,
                       idx_v, minv_v, rows_v0, rows_v1, out_v,
                       sem0, sem1):
    sc = lax.axis_index("c")
    sub = lax.axis_index("s")
    gbase = sc * GPS + sub * GPW
    rowsv = (rows_v0, rows_v1)
    sems = (sem0, sem1)

    pltpu.sync_copy(idx_hbm.at[pl.ds(gbase * 2, 2 * GPW)], idx_v)
    pltpu.sync_copy(minv_hbm.at[pl.ds(gbase, GPW)], minv_v)

    def start(c, par):
        return pltpu.async_copy(
            tab.at[idx_v.at[pl.ds(c * 2 * G, 2 * G)]], rowsv[par], sems[par])

    def do_chunk(c, par):
        base = gbase + c * G
        pltpu.make_async_copy(
            tab.at[idx_v.at[pl.ds(c * 2 * G, 2 * G)]], rowsv[par],
            sems[par]).wait()
        rows = rowsv[par]

        def gate(g, carry2):
            m = minv_v[c * G + g]
            for cc in range(W // 16):
                a = rows[2 * g, cc * 16:(cc + 1) * 16]
                b = rows[2 * g + 1, cc * 16:(cc + 1) * 16]
                out_v[g, cc * 16:(cc + 1) * 16] = ~((a & b) ^ (m & (a ^ b)))
            return carry2

        lax.fori_loop(0, G, gate, 0)
        pltpu.sync_copy(out_v, out.at[pl.ds(base, G)])

    start(0, 0)
    for c in range(NCH):
        if c + 1 < NCH:
            start(c + 1, (c + 1) % 2)
        do_chunk(c, c % 2)


def _sc_gather_layer(table, idx, minv_sp):
    mesh = plsc.VectorSubcoreMesh(core_axis_name="c", subcore_axis_name="s",
                                  num_cores=2, num_subcores=16)
    return pl.kernel(
        _gather_layer_body,
        out_type=jax.ShapeDtypeStruct((GATES, W), jnp.int32),
        mesh=mesh,
        scratch_types=[
            pltpu.VMEM((2 * GPW,), jnp.int32),
            pltpu.VMEM((GPW, 16), jnp.int32),
            pltpu.VMEM((2 * G, W), jnp.int32),
            pltpu.VMEM((2 * G, W), jnp.int32),
            pltpu.VMEM((G, W), jnp.int32),
            pltpu.SemaphoreType.DMA,
            pltpu.SemaphoreType.DMA,
        ],
    )(table, idx, minv_sp)


def kernel(input_bitarrays, batch_size,
           adj_logits_0, invert_logits_0, adj_logits_1, invert_logits_1,
           adj_logits_2, invert_logits_2, adj_logits_3, invert_logits_3):
    adjs = (adj_logits_0, adj_logits_1, adj_logits_2, adj_logits_3)
    vs = (invert_logits_0, invert_logits_1, invert_logits_2, invert_logits_3)

    rkey = jax.random.key(42)
    kc = [jax.random.key_data(jax.random.fold_in(rkey, 2 * i))
          for i in range(NL)]
    kb = [jax.random.key_data(jax.random.fold_in(rkey, 2 * i + 1))
          for i in range(NL)]
    keys_c = jnp.stack(kc).astype(jnp.uint32)
    keys_b = jnp.stack(kb).astype(jnp.uint32)

    p_stack = jnp.stack([jax.nn.sigmoid(v) for v in vs]).reshape(NL, 1, N)

    samples = [_sample_layer(keys_c[l], adjs[l].reshape(R2, N))
               for l in range(NL)]
    minv = _bern_all(keys_b, p_stack)

    conns = []
    for l in range(NL):
        s = jnp.transpose(samples[l])
        conn = jnp.transpose(s.reshape(B, 2, N), (0, 2, 1))
        conns.append(conn)
    invs = [minv[l] != 0 for l in range(NL)]

    boff = (jnp.arange(B, dtype=jnp.int32) * N)[:, None, None]
    table = input_bitarrays + (batch_size - B)
    for l in range(NL):
        idx = (conns[l] if l == 0 else conns[l] + boff).reshape(2 * GATES)
        minv_sp = jnp.broadcast_to(minv[l].reshape(GATES, 1), (GATES, 16))
        table = _sc_gather_layer(table, idx, minv_sp)

    x = table.reshape(B, N, W)
    return (x,) + tuple(conns) + tuple(invs)
